# Initial kernel scaffold; baseline (speedup 1.0000x reference)
import jax
import jax.numpy as jnp
from jax.experimental import pallas as pl


def kernel(x, edge_index, batch, l0_W1, l0_b1, l0_W2, l0_b2, l0_eps, l1_W1, l1_b1, l1_W2, l1_b2, l1_eps, l2_W1, l2_b1, l2_W2, l2_b2, l2_eps, d_W1, d_b1, bn_gamma, bn_beta, d_W2, d_b2):
    raise NotImplementedError("write your pallas kernel here")


# speedup vs baseline: 3.0054x; 3.0054x over previous
import functools

import jax
import jax.numpy as jnp
from jax import lax
from jax.experimental import pallas as pl
from jax.experimental.pallas import tpu as pltpu
from jax.experimental.pallas import tpu_sc as plsc

N = 10000
E = 160000
D = 256
G = 64

NC = 2
NS = 16
CH = 128
K = 79
EPT = K * CH
EPAD = NS * EPT
TRASH = NS
ACC_ROWS = N + TRASH
ROWS_PER_TILE = N // NS
ZROWS = ACC_ROWS // NS

R = 1000
GRID = N // R


def _sc_scatter_body(h3, src2, dstx, mfl, out3,
                     src_v, dst_v, rows_v, m_v, acc3, sem):
    c = lax.axis_index("c")
    s = lax.axis_index("s")

    pltpu.sync_copy(src2.at[c, s], src_v)
    pltpu.sync_copy(dstx.at[s], dst_v)
    pltpu.sync_copy(mfl.at[s], m_v)

    def _zrow(i, _):
        for v in range(8):
            rows_v[i, 0, pl.ds(v * 16, 16)] = jnp.zeros((16,), jnp.float32)
        return 0
    lax.fori_loop(0, CH, _zrow, 0)
    zbase = s * ZROWS
    off = 0
    while off < ZROWS:
        nr = min(CH, ZROWS - off)
        pltpu.sync_copy(rows_v.at[pl.ds(0, nr)], acc3.at[pl.ds(zbase + off, nr)])
        off += nr
    plsc.subcore_barrier()

    zero16 = jnp.zeros((16,), jnp.float32)

    def _chunk(j, prev8):
        pltpu.async_copy(h3.at[src_v.at[j]], rows_v, sem).wait()

        def _grp(g, prev8):
            mg = m_v[j, pl.ds(g * 16, 16)]
            for r in range(16):
                m = mg[r]
                i = g * 16 + r
                new = []
                for v in range(8):
                    cur = rows_v[i, 0, pl.ds(v * 16, 16)] + m * prev8[v]
                    rows_v[i, 0, pl.ds(v * 16, 16)] = cur
                    new.append(cur)
                prev8 = tuple(new)
            return prev8

        prev8 = lax.fori_loop(0, CH // 16, _grp, prev8)
        pltpu.sync_copy(rows_v, acc3.at[dst_v.at[j]], add=True)
        return prev8

    lax.fori_loop(0, K, _chunk, (zero16,) * 8)
    plsc.subcore_barrier()

    wbase = s * ROWS_PER_TILE
    pltpu.sync_copy(acc3.at[pl.ds(wbase, ROWS_PER_TILE)],
                    out3.at[pl.ds(wbase, ROWS_PER_TILE), pl.ds(c, 1)])


def _make_sc_scatter():
    mesh = plsc.VectorSubcoreMesh(core_axis_name="c", subcore_axis_name="s")
    return pl.kernel(
        _sc_scatter_body,
        out_type=jax.ShapeDtypeStruct((N, NC, 128), jnp.float32),
        mesh=mesh,
        scratch_types=[
            pltpu.VMEM((K, CH), jnp.int32),
            pltpu.VMEM((K, CH), jnp.int32),
            pltpu.VMEM((CH, 1, 128), jnp.float32),
            pltpu.VMEM((K, CH), jnp.float32),
            pltpu.VMEM_SHARED((ACC_ROWS, 1, 128), jnp.float32),
            pltpu.SemaphoreType.DMA,
        ],
    )


def _mlp_body(eps_ref, batch_ref, h_ref, agg_ref, w1_ref, b1_ref, w2_ref,
              b2_ref, out_ref, pooled_ref):
    i = pl.program_id(0)
    h = h_ref[...]
    h_in = (1.0 + eps_ref[0, 0]) * h + agg_ref[...]
    t = jnp.dot(h_in, w1_ref[...], preferred_element_type=jnp.float32)
    t = jnp.maximum(t + b1_ref[...], 0.0)
    out = jnp.dot(t, w2_ref[...], preferred_element_type=jnp.float32)
    out = out + b2_ref[...] + h
    out_ref[...] = out

    ids = batch_ref[0]
    onehot_t = (lax.broadcasted_iota(jnp.int32, (G, R), 0) == ids
                ).astype(jnp.float32)
    pb = jnp.dot(onehot_t, out, preferred_element_type=jnp.float32,
                 precision=lax.Precision.HIGHEST)

    @pl.when(i == 0)
    def _():
        pooled_ref[...] = jnp.zeros_like(pooled_ref)
    pooled_ref[...] += pb


def _mlp(eps2, batch3, h, agg, w1, b1, w2, b2):
    return pl.pallas_call(
        _mlp_body,
        grid=(GRID,),
        in_specs=[
            pl.BlockSpec(memory_space=pltpu.SMEM),
            pl.BlockSpec((1, 1, R), lambda i: (i, 0, 0)),
            pl.BlockSpec((R, D), lambda i: (i, 0)),
            pl.BlockSpec((R, D), lambda i: (i, 0)),
            pl.BlockSpec((D, D), lambda i: (0, 0)),
            pl.BlockSpec((1, D), lambda i: (0, 0)),
            pl.BlockSpec((D, D), lambda i: (0, 0)),
            pl.BlockSpec((1, D), lambda i: (0, 0)),
        ],
        out_specs=[
            pl.BlockSpec((R, D), lambda i: (i, 0)),
            pl.BlockSpec((G, D), lambda i: (0, 0)),
        ],
        out_shape=[
            jax.ShapeDtypeStruct((N, D), jnp.float32),
            jax.ShapeDtypeStruct((G, D), jnp.float32),
        ],
        compiler_params=pltpu.CompilerParams(
            dimension_semantics=("arbitrary",),
        ),
    )(eps2, batch3, h, agg, w1, b1, w2, b2)


def _head_body(p0_ref, p1_ref, p2_ref, w1_ref, b1_ref, g_ref, be_ref,
               w2_ref, b2_ref, out_ref):
    hcat = jnp.concatenate([p0_ref[...], p1_ref[...], p2_ref[...]], axis=1)
    z = jnp.dot(hcat, w1_ref[...], preferred_element_type=jnp.float32)
    z = z + b1_ref[...]
    mean = jnp.mean(z, axis=0, keepdims=True)
    zc = z - mean
    var = jnp.mean(zc * zc, axis=0, keepdims=True)
    zn = zc / jnp.sqrt(var + 1e-5) * g_ref[...] + be_ref[...]
    zn = jnp.maximum(zn, 0.0)
    out = jnp.dot(zn, w2_ref[...], preferred_element_type=jnp.float32)
    out_ref[...] = out + b2_ref[...]


def _head(p0, p1, p2, w1, b1, gamma, beta, w2, b2):
    return pl.pallas_call(
        _head_body,
        out_shape=jax.ShapeDtypeStruct((G, D), jnp.float32),
    )(p0, p1, p2, w1, b1, gamma, beta, w2, b2)


def kernel(x, edge_index, batch,
           l0_W1, l0_b1, l0_W2, l0_b2, l0_eps,
           l1_W1, l1_b1, l1_W2, l1_b2, l1_eps,
           l2_W1, l2_b1, l2_W2, l2_b2, l2_eps,
           d_W1, d_b1, bn_gamma, bn_beta, d_W2, d_b2):
    perm = jnp.argsort(edge_index[1], stable=True)
    src = edge_index[0][perm]
    dst = edge_index[1][perm]
    pad = EPAD - E
    pad_i = jnp.arange(pad, dtype=jnp.int32)
    src_p = jnp.concatenate([src, pad_i % N])
    dst_p = jnp.concatenate([dst, jnp.full((pad,), -1, jnp.int32)])
    d2 = dst_p.reshape(NS, EPT)
    cont = jnp.concatenate(
        [jnp.zeros((NS, 1), bool), d2[:, 1:] == d2[:, :-1]], axis=1)
    runend = jnp.concatenate(
        [d2[:, 1:] != d2[:, :-1], jnp.ones((NS, 1), bool)], axis=1)
    pos = jnp.broadcast_to(jnp.arange(EPT, dtype=jnp.int32) % NS, (NS, EPT))
    dstx = jnp.where(runend & (d2 >= 0), d2, N + pos).reshape(NS, K, CH)
    mfl = cont.astype(jnp.float32).reshape(NS, K, CH)
    src2 = jnp.stack([2 * src_p, 2 * src_p + 1]).reshape(NC, NS, K, CH)
    batch3 = batch.reshape(GRID, 1, R)

    sc_scatter = _make_sc_scatter()

    layer_params = [
        (l0_W1, l0_b1, l0_W2, l0_b2, l0_eps),
        (l1_W1, l1_b1, l1_W2, l1_b2, l1_eps),
        (l2_W1, l2_b1, l2_W2, l2_b2, l2_eps),
    ]
    h = x
    pooled = []
    for (W1, b1, W2, b2, eps) in layer_params:
        agg3 = sc_scatter(h.reshape(NC * N, 1, 128), src2, dstx, mfl)
        h, p = _mlp(eps.reshape(1, 1), batch3, h, agg3.reshape(N, D),
                    W1, b1.reshape(1, D), W2, b2.reshape(1, D))
        pooled.append(p)

    graph_emb = _head(pooled[0], pooled[1], pooled[2],
                      d_W1, d_b1.reshape(1, 3 * D), bn_gamma.reshape(1, 3 * D),
                      bn_beta.reshape(1, 3 * D), d_W2, d_b2.reshape(1, D))
    return (h, graph_emb)
